# Initial kernel scaffold; baseline (speedup 1.0000x reference)
import jax
import jax.numpy as jnp
from jax.experimental import pallas as pl


def kernel(x, edge_index, edge_attr, batch_idx, m0_w1, m0_b1, m0_w2, m0_b2, u0_w1, u0_b1, u0_w2, u0_b2, m1_w1, m1_b1, m1_w2, m1_b2, u1_w1, u1_b1, u1_w2, u1_b2, lin1_w, lin1_b, lin2_w, lin2_b):
    raise NotImplementedError("write your pallas kernel here")


# speedup vs baseline: 3.4021x; 3.4021x over previous
import functools

import jax
import jax.numpy as jnp
from jax import lax
from jax.experimental import pallas as pl
from jax.experimental.pallas import tpu as pltpu
from jax.experimental.pallas import tpu_sc as plsc

_NC = 2
_NS = 16
_NW = _NC * _NS
_G = 64


def _mm_body(x_ref, w_ref, b_ref, o_ref):
    o_ref[...] = (
        jnp.dot(x_ref[...], w_ref[...], preferred_element_type=jnp.float32)
        + b_ref[...]
    )


def _mm(x, w, b, block_m):
    m, k = x.shape
    f = w.shape[1]
    assert m % block_m == 0
    return pl.pallas_call(
        _mm_body,
        grid=(m // block_m,),
        in_specs=[
            pl.BlockSpec((block_m, k), lambda i: (i, 0)),
            pl.BlockSpec((k, f), lambda i: (0, 0)),
            pl.BlockSpec((1, f), lambda i: (0, 0)),
        ],
        out_specs=pl.BlockSpec((block_m, f), lambda i: (i, 0)),
        out_shape=jax.ShapeDtypeStruct((m, f), jnp.float32),
    )(x, w, b.reshape(1, f))


def _update_body(acc_ref, degt_ref, h_ref, w2_ref, b2_ref, uw1h_ref, uw1a_ref,
                 ub1_ref, uw2_ref, ub2_ref, o_ref):
    acc = acc_ref[0] + acc_ref[1]
    deg = jnp.sum(degt_ref[...], axis=1, keepdims=True)
    agg = (jnp.dot(acc, w2_ref[...], preferred_element_type=jnp.float32)
           + deg * b2_ref[...])
    u = jnp.maximum(
        jnp.dot(h_ref[...], uw1h_ref[...], preferred_element_type=jnp.float32)
        + jnp.dot(agg, uw1a_ref[...], preferred_element_type=jnp.float32)
        + ub1_ref[...], 0.0)
    h2 = jnp.dot(u, uw2_ref[...], preferred_element_type=jnp.float32) + ub2_ref[...]
    o_ref[...] = jnp.maximum(h2, 0.0)


def _update(acc, degt, h, w2, b2, uw1h, uw1a, ub1, uw2, ub2, block_m):
    n, d = h.shape
    f = uw2.shape[1]
    return pl.pallas_call(
        _update_body,
        grid=(n // block_m,),
        in_specs=[
            pl.BlockSpec((2, block_m, d), lambda i: (0, i, 0)),
            pl.BlockSpec((block_m, _NC), lambda i: (i, 0)),
            pl.BlockSpec((block_m, d), lambda i: (i, 0)),
            pl.BlockSpec((d, f), lambda i: (0, 0)),
            pl.BlockSpec((1, f), lambda i: (0, 0)),
            pl.BlockSpec((d, f), lambda i: (0, 0)),
            pl.BlockSpec((d, f), lambda i: (0, 0)),
            pl.BlockSpec((1, f), lambda i: (0, 0)),
            pl.BlockSpec((d, f), lambda i: (0, 0)),
            pl.BlockSpec((1, f), lambda i: (0, 0)),
        ],
        out_specs=pl.BlockSpec((block_m, f), lambda i: (i, 0)),
        out_shape=jax.ShapeDtypeStruct((n, f), jnp.float32),
    )(acc, degt, h, w2, b2.reshape(1, f), uw1h, uw1a, ub1.reshape(1, f),
      uw2, ub2.reshape(1, f))


def _head_body(h_ref, bidx_ref, lw1_ref, lb1_ref, lw2_ref, lb2_ref, o_ref):
    bidx = bidx_ref[...]
    gids = lax.broadcasted_iota(jnp.int32, (_G, 1), 0)
    onehot = (bidx == gids).astype(jnp.float32)
    sums = jnp.dot(onehot, h_ref[...], preferred_element_type=jnp.float32)
    counts = jnp.sum(onehot, axis=1, keepdims=True)
    pooled = sums / jnp.maximum(counts, 1.0)
    h2 = jnp.maximum(
        jnp.dot(pooled, lw1_ref[...], preferred_element_type=jnp.float32)
        + lb1_ref[...], 0.0)
    o_ref[...] = jnp.sum(h2 * lw2_ref[...], axis=1, keepdims=True) + lb2_ref[...]


def _head(h, batch_idx, lw1, lb1, lw2, lb2):
    n, d = h.shape
    return pl.pallas_call(
        _head_body,
        out_shape=jax.ShapeDtypeStruct((_G, 1), jnp.float32),
    )(h, batch_idx.reshape(1, n), lw1, lb1.reshape(1, d),
      lw2.reshape(1, d), lb2.reshape(1, 1))


def _make_edge_kernel(n, e, d, chunk):
    e_per_w = e // _NW
    steps = e_per_w // chunk
    assert e % _NW == 0 and e_per_w % chunk == 0 and chunk % 16 == 0
    assert chunk <= 128
    rows_per_tile = (n // _NS) // 8 * 8
    tail_base = rows_per_tile * _NS
    tail = n - tail_base
    zr = 16
    assert rows_per_tile % zr == 0 and tail % 8 == 0 and tail <= zr
    mesh = plsc.VectorSubcoreMesh(core_axis_name="c", subcore_axis_name="s",
                                  num_cores=_NC, num_subcores=_NS)

    @functools.partial(
        pl.kernel,
        out_type=[
            jax.ShapeDtypeStruct((_NC, n, d), jnp.float32),
            jax.ShapeDtypeStruct((_NC * n,), jnp.float32),
        ],
        mesh=mesh,
        scratch_types=[
            pltpu.VMEM((chunk,), jnp.int32),
            pltpu.VMEM((chunk,), jnp.int32),
            pltpu.VMEM((chunk, d), jnp.float32),
            pltpu.VMEM((chunk, d), jnp.float32),
            pltpu.VMEM((chunk, d), jnp.float32),
            pltpu.VMEM((zr, d), jnp.float32),
            pltpu.VMEM((rows_per_tile + tail,), jnp.float32),
            pltpu.VMEM((chunk,), jnp.float32),
            pltpu.VMEM_SHARED((n, d), jnp.float32),
            pltpu.VMEM_SHARED((n,), jnp.float32),
            pltpu.SemaphoreType.DMA,
            pltpu.SemaphoreType.DMA,
        ],
    )
    def edge_kernel(pd_hbm, ps_hbm, pe_hbm, dst_hbm, src_hbm,
                    acc_hbm, deg_hbm,
                    idx_d, idx_s, pd_v, ps_v, pe_v, zbuf, zbuf1, ones_v,
                    acc_sh, deg_sh, sem_d, sem_s):
        cid = lax.axis_index("c")
        sid = lax.axis_index("s")
        wid = sid * _NC + cid

        zero16 = jnp.zeros((16,), jnp.float32)
        ones16 = jnp.ones((16,), jnp.float32)

        def zrow(i, _):
            def zcol(k, _):
                zbuf[i, pl.ds(k * 16, 16)] = zero16
                return 0
            return lax.fori_loop(0, d // 16, zcol, 0)
        lax.fori_loop(0, zr, zrow, 0)

        def zones(i, _):
            ones_v[pl.ds(i * 16, 16)] = ones16
            return 0
        lax.fori_loop(0, chunk // 16, zones, 0)

        def zvec(i, _):
            zbuf1[pl.ds(i * 16, 16)] = zero16
            return 0
        lax.fori_loop(0, (rows_per_tile + tail) // 16, zvec, 0)

        base_row = sid * rows_per_tile
        for q in range(rows_per_tile // zr):
            pltpu.sync_copy(zbuf, acc_sh.at[pl.ds(base_row + q * zr, zr)])

        @pl.when(sid < _NS - 1)
        def _zero_deg():
            pltpu.sync_copy(zbuf1.at[pl.ds(0, rows_per_tile)],
                            deg_sh.at[pl.ds(base_row, rows_per_tile)])

        @pl.when(sid == _NS - 1)
        def _zero_tail():
            pltpu.sync_copy(zbuf.at[pl.ds(0, tail)],
                            acc_sh.at[pl.ds(tail_base, tail)])
            pltpu.sync_copy(zbuf1,
                            deg_sh.at[pl.ds(base_row, rows_per_tile + tail)])
        plsc.subcore_barrier()

        def step(j, _):
            base = wid * e_per_w + j * chunk
            pltpu.sync_copy(dst_hbm.at[pl.ds(base, chunk)], idx_d)
            pltpu.sync_copy(src_hbm.at[pl.ds(base, chunk)], idx_s)
            cp_d = pltpu.async_copy(pd_hbm.at[idx_d], pd_v, sem_d)
            cp_s = pltpu.async_copy(ps_hbm.at[idx_s], ps_v, sem_s)
            pltpu.sync_copy(pe_hbm.at[pl.ds(base, chunk)], pe_v)
            pltpu.sync_copy(ones_v, deg_sh.at[idx_d], add=True)

            cp_d.wait()
            cp_s.wait()

            def row(i, _):
                def col(k, _):
                    s = pl.ds(k * 16, 16)
                    v = pd_v[i, s] + ps_v[i, s] + pe_v[i, s]
                    pd_v[i, s] = jnp.maximum(v, 0.0)
                    return 0
                return lax.fori_loop(0, d // 16, col, 0)
            lax.fori_loop(0, chunk, row, 0)

            pltpu.sync_copy(pd_v, acc_sh.at[idx_d], add=True)
            return 0
        lax.fori_loop(0, steps, step, 0)

        plsc.subcore_barrier()
        pltpu.sync_copy(acc_sh.at[pl.ds(base_row, rows_per_tile)],
                        acc_hbm.at[cid, pl.ds(base_row, rows_per_tile)])

        @pl.when(sid == _NS - 1)
        def _write_tail():
            pltpu.sync_copy(acc_sh.at[pl.ds(tail_base, tail)],
                            acc_hbm.at[cid, pl.ds(tail_base, tail)])
            pltpu.sync_copy(deg_sh.at[pl.ds(base_row, rows_per_tile + tail)],
                            zbuf1)
            pltpu.sync_copy(zbuf1,
                            deg_hbm.at[pl.ds(cid * n + base_row,
                                             rows_per_tile + tail)])

        @pl.when(sid < _NS - 1)
        def _write_deg():
            pltpu.sync_copy(deg_sh.at[pl.ds(base_row, rows_per_tile)],
                            zbuf1.at[pl.ds(0, rows_per_tile)])
            pltpu.sync_copy(
                zbuf1.at[pl.ds(0, rows_per_tile)],
                deg_hbm.at[pl.ds(cid * n + base_row, rows_per_tile)])

    return edge_kernel


def kernel(x, edge_index, edge_attr, batch_idx,
           m0_w1, m0_b1, m0_w2, m0_b2, u0_w1, u0_b1, u0_w2, u0_b2,
           m1_w1, m1_b1, m1_w2, m1_b2, u1_w1, u1_b1, u1_w2, u1_b2,
           lin1_w, lin1_b, lin2_w, lin2_b):
    n, d = x.shape
    e = edge_index.shape[1]
    h_dim = m0_w2.shape[0]
    src = edge_index[0]
    dst = edge_index[1]

    edge_fn = _make_edge_kernel(n, e, h_dim, chunk=80)

    def conv(h, mw1, mb1, mw2, mb2, uw1, ub1, uw2, ub2, degt):
        hd = h.shape[1]
        pd = _mm(h, mw1[:hd], mb1 * 0.0, block_m=2000)
        ps = _mm(h, mw1[hd:2 * hd], mb1 * 0.0, block_m=2000)
        pe = _mm(edge_attr, mw1[2 * hd:], mb1, block_m=8000)
        acc, deg = edge_fn(pd, ps, pe, dst, src)
        if degt is None:
            degt = jnp.transpose(deg.reshape(_NC, h.shape[0]))
        h_new = _update(acc, degt, h, mw2, mb2, uw1[:hd], uw1[hd:],
                        ub1, uw2, ub2, block_m=2000)
        return h_new, degt

    h1, degt = conv(x, m0_w1, m0_b1, m0_w2, m0_b2,
                    u0_w1, u0_b1, u0_w2, u0_b2, None)
    h2, _ = conv(h1, m1_w1, m1_b1, m1_w2, m1_b2,
                 u1_w1, u1_b1, u1_w2, u1_b2, degt)
    return _head(h2, batch_idx, lin1_w, lin1_b, lin2_w, lin2_b)
